# baseline (speedup 1.0000x reference)
import jax
import jax.numpy as jnp
from jax.experimental import pallas as pl
from jax.experimental.pallas import tpu as pltpu

_NE = 8
_D = 768
_H = 12
_DH = 64
_T = 512
_DE = _NE * _D


def _block(x_ref, Wr_ref, br_ref, We_ref, be_ref, Wq_ref, bq_ref,
           Wk_ref, bk_ref, Wv_ref, bv_ref, Wo_ref, bo_ref, o_ref):
    x = x_ref[...]
    t = x.shape[0]
    f32 = jnp.float32

    logits = jnp.dot(x, Wr_ref[...]) + br_ref[...]
    lm = jnp.max(logits, axis=1, keepdims=True)
    ex = jnp.exp(logits - lm)
    probs = ex / jnp.sum(ex, axis=1, keepdims=True)

    col = jax.lax.broadcasted_iota(jnp.int32, (t, _NE), 1)
    p1 = jnp.max(probs, axis=1, keepdims=True)
    i1 = jnp.min(jnp.where(probs == p1, col, _NE), axis=1, keepdims=True)
    probs_m = jnp.where(col == i1, -jnp.inf, probs)
    p2 = jnp.max(probs_m, axis=1, keepdims=True)
    i2 = jnp.min(jnp.where(probs_m == p2, col, _NE), axis=1, keepdims=True)
    gate = probs * ((col == i1) | (col == i2)).astype(f32)

    xb = x.astype(jnp.bfloat16)
    z = jax.lax.dot(xb, We_ref[...], preferred_element_type=f32)
    h = jnp.maximum(z + be_ref[...], 0.0)
    blk = jax.lax.broadcasted_iota(jnp.int32, (_NE, _DE), 1) // _D
    row = jax.lax.broadcasted_iota(jnp.int32, (_NE, _DE), 0)
    gate_b = jnp.dot(gate, (blk == row).astype(f32))
    g = h * gate_b
    parts = [g[:, e * _D:(e + 1) * _D] for e in range(_NE)]
    while len(parts) > 1:
        parts = [parts[i] + parts[i + 1] for i in range(0, len(parts), 2)]
    acc = parts[0]

    ab = acc.astype(jnp.bfloat16)
    q = jax.lax.dot(ab, Wq_ref[...], preferred_element_type=f32) + bq_ref[...]
    k = jax.lax.dot(ab, Wk_ref[...], preferred_element_type=f32) + bk_ref[...]
    v = jax.lax.dot(ab, Wv_ref[...], preferred_element_type=f32) + bv_ref[...]
    seg = (jax.lax.broadcasted_iota(jnp.int32, (_D, _H), 0) // _DH ==
           jax.lax.broadcasted_iota(jnp.int32, (_D, _H), 1))
    S = seg.astype(f32)
    s = jnp.dot(q * k, S) * (1.0 / 8.0)
    sm = jnp.max(s, axis=1, keepdims=True)
    se = jnp.exp(s - sm)
    w = se / jnp.sum(se, axis=1, keepdims=True)
    wb = jnp.dot(w, S.T)
    y = jax.lax.dot((wb * v).astype(jnp.bfloat16), Wo_ref[...],
                    preferred_element_type=f32) + bo_ref[...]
    o_ref[...] = y


def kernel(inputs, Wr, br, We, be, Wq, bq, Wk, bk, Wv, bv, Wo, bo):
    n = inputs.shape[0]
    br2 = br.reshape(1, _NE)
    bq2 = bq.reshape(1, _D)
    bk2 = bk.reshape(1, _D)
    bv2 = bv.reshape(1, _D)
    bo2 = bo.reshape(1, _D)
    bf16 = jnp.bfloat16
    We_cat = We.transpose(1, 0, 2).reshape(_D, _DE).astype(bf16)
    be_cat = be.reshape(1, _DE)
    const = lambda *zeros: (lambda i: zeros)
    return pl.pallas_call(
        _block,
        grid=(n // _T,),
        in_specs=[
            pl.BlockSpec((_T, _D), lambda i: (i, 0)),
            pl.BlockSpec((_D, _NE), const(0, 0)),
            pl.BlockSpec((1, _NE), const(0, 0)),
            pl.BlockSpec((_D, _DE), const(0, 0)),
            pl.BlockSpec((1, _DE), const(0, 0)),
            pl.BlockSpec((_D, _D), const(0, 0)),
            pl.BlockSpec((1, _D), const(0, 0)),
            pl.BlockSpec((_D, _D), const(0, 0)),
            pl.BlockSpec((1, _D), const(0, 0)),
            pl.BlockSpec((_D, _D), const(0, 0)),
            pl.BlockSpec((1, _D), const(0, 0)),
            pl.BlockSpec((_D, _D), const(0, 0)),
            pl.BlockSpec((1, _D), const(0, 0)),
        ],
        out_specs=pl.BlockSpec((_T, _D), lambda i: (i, 0)),
        out_shape=jax.ShapeDtypeStruct((n, _D), jnp.float32),
        compiler_params=pltpu.CompilerParams(
            dimension_semantics=("parallel",)),
    )(inputs, Wr, br2, We_cat, be_cat, Wq.astype(bf16), bq2,
      Wk.astype(bf16), bk2, Wv.astype(bf16), bv2, Wo.astype(bf16), bo2)


# speedup vs baseline: 1.3163x; 1.3163x over previous
import jax
import jax.numpy as jnp
from jax.experimental import pallas as pl
from jax.experimental.pallas import tpu as pltpu

_NE = 8
_D = 768
_H = 12
_DH = 64
_T = 512


def _block(x_ref, Wr_ref, br_ref, We_ref, be_ref, Wq_ref, bq_ref,
           Wk_ref, bk_ref, Wv_ref, bv_ref, Wo_ref, bo_ref, o_ref):
    x = x_ref[...]
    t = x.shape[0]
    f32 = jnp.float32

    logits = jnp.dot(x, Wr_ref[...]) + br_ref[...]
    lm = jnp.max(logits, axis=1, keepdims=True)
    ex = jnp.exp(logits - lm)
    probs = ex / jnp.sum(ex, axis=1, keepdims=True)

    col = jax.lax.broadcasted_iota(jnp.int32, (t, _NE), 1)
    p1 = jnp.max(probs, axis=1, keepdims=True)
    i1 = jnp.min(jnp.where(probs == p1, col, _NE), axis=1, keepdims=True)
    probs_m = jnp.where(col == i1, -jnp.inf, probs)
    p2 = jnp.max(probs_m, axis=1, keepdims=True)
    i2 = jnp.min(jnp.where(probs_m == p2, col, _NE), axis=1, keepdims=True)
    gate = probs * ((col == i1) | (col == i2)).astype(f32)

    acc = jnp.zeros((t, _D), f32)
    for e in range(_NE):
        z = jax.lax.dot(x, We_ref[e], preferred_element_type=f32)
        h = jnp.maximum(z + be_ref[e:e + 1, :], 0.0)
        acc = acc + gate[:, e:e + 1] * h

    q = jax.lax.dot(acc, Wq_ref[...], preferred_element_type=f32) + bq_ref[...]
    k = jax.lax.dot(acc, Wk_ref[...], preferred_element_type=f32) + bk_ref[...]
    v = jax.lax.dot(acc, Wv_ref[...], preferred_element_type=f32) + bv_ref[...]
    seg = (jax.lax.broadcasted_iota(jnp.int32, (_D, _H), 0) // _DH ==
           jax.lax.broadcasted_iota(jnp.int32, (_D, _H), 1))
    S = seg.astype(f32)
    s = jnp.dot(q * k, S) * (1.0 / 8.0)
    sm = jnp.max(s, axis=1, keepdims=True)
    se = jnp.exp(s - sm)
    w = se / jnp.sum(se, axis=1, keepdims=True)
    wb = jnp.dot(w, S.T)
    y = jax.lax.dot(wb * v, Wo_ref[...],
                    preferred_element_type=f32) + bo_ref[...]
    o_ref[...] = y


def kernel(inputs, Wr, br, We, be, Wq, bq, Wk, bk, Wv, bv, Wo, bo):
    n = inputs.shape[0]
    br2 = br.reshape(1, _NE)
    bq2 = bq.reshape(1, _D)
    bk2 = bk.reshape(1, _D)
    bv2 = bv.reshape(1, _D)
    bo2 = bo.reshape(1, _D)
    const = lambda *zeros: (lambda i: zeros)
    return pl.pallas_call(
        _block,
        grid=(n // _T,),
        in_specs=[
            pl.BlockSpec((_T, _D), lambda i: (i, 0)),
            pl.BlockSpec((_D, _NE), const(0, 0)),
            pl.BlockSpec((1, _NE), const(0, 0)),
            pl.BlockSpec((_NE, _D, _D), const(0, 0, 0)),
            pl.BlockSpec((_NE, _D), const(0, 0)),
            pl.BlockSpec((_D, _D), const(0, 0)),
            pl.BlockSpec((1, _D), const(0, 0)),
            pl.BlockSpec((_D, _D), const(0, 0)),
            pl.BlockSpec((1, _D), const(0, 0)),
            pl.BlockSpec((_D, _D), const(0, 0)),
            pl.BlockSpec((1, _D), const(0, 0)),
            pl.BlockSpec((_D, _D), const(0, 0)),
            pl.BlockSpec((1, _D), const(0, 0)),
        ],
        out_specs=pl.BlockSpec((_T, _D), lambda i: (i, 0)),
        out_shape=jax.ShapeDtypeStruct((n, _D), jnp.float32),
        compiler_params=pltpu.CompilerParams(
            dimension_semantics=("parallel",)),
    )(inputs, Wr, br2, We, be, Wq, bq2,
      Wk, bk2, Wv, bv2, Wo, bo2)


# speedup vs baseline: 1.3659x; 1.0377x over previous
import jax
import jax.numpy as jnp
from jax.experimental import pallas as pl
from jax.experimental.pallas import tpu as pltpu

_NE = 8
_D = 768
_H = 12
_DH = 64
_T = 1024


def _block(x_ref, Wr_ref, br_ref, We_ref, be_ref, Wq_ref, bq_ref,
           Wk_ref, bk_ref, Wv_ref, bv_ref, Wo_ref, bo_ref, o_ref):
    x = x_ref[...]
    t = x.shape[0]
    f32 = jnp.float32

    logits = jnp.dot(x, Wr_ref[...]) + br_ref[...]
    lm = jnp.max(logits, axis=1, keepdims=True)
    ex = jnp.exp(logits - lm)
    probs = ex / jnp.sum(ex, axis=1, keepdims=True)

    col = jax.lax.broadcasted_iota(jnp.int32, (t, _NE), 1)
    p1 = jnp.max(probs, axis=1, keepdims=True)
    i1 = jnp.min(jnp.where(probs == p1, col, _NE), axis=1, keepdims=True)
    probs_m = jnp.where(col == i1, -jnp.inf, probs)
    p2 = jnp.max(probs_m, axis=1, keepdims=True)
    i2 = jnp.min(jnp.where(probs_m == p2, col, _NE), axis=1, keepdims=True)
    gate = probs * ((col == i1) | (col == i2)).astype(f32)

    acc = jnp.zeros((t, _D), f32)
    for e in range(_NE):
        z = jax.lax.dot(x, We_ref[e], preferred_element_type=f32)
        h = jnp.maximum(z + be_ref[e:e + 1, :], 0.0)
        acc = acc + gate[:, e:e + 1] * h

    q = jax.lax.dot(acc, Wq_ref[...], preferred_element_type=f32) + bq_ref[...]
    k = jax.lax.dot(acc, Wk_ref[...], preferred_element_type=f32) + bk_ref[...]
    v = jax.lax.dot(acc, Wv_ref[...], preferred_element_type=f32) + bv_ref[...]
    seg = (jax.lax.broadcasted_iota(jnp.int32, (_D, _H), 0) // _DH ==
           jax.lax.broadcasted_iota(jnp.int32, (_D, _H), 1))
    S = seg.astype(f32)
    s = jnp.dot(q * k, S) * (1.0 / 8.0)
    sm = jnp.max(s, axis=1, keepdims=True)
    se = jnp.exp(s - sm)
    w = se / jnp.sum(se, axis=1, keepdims=True)
    wb = jnp.dot(w, S.T)
    y = jax.lax.dot(wb * v, Wo_ref[...],
                    preferred_element_type=f32) + bo_ref[...]
    o_ref[...] = y


def kernel(inputs, Wr, br, We, be, Wq, bq, Wk, bk, Wv, bv, Wo, bo):
    n = inputs.shape[0]
    br2 = br.reshape(1, _NE)
    bq2 = bq.reshape(1, _D)
    bk2 = bk.reshape(1, _D)
    bv2 = bv.reshape(1, _D)
    bo2 = bo.reshape(1, _D)
    const = lambda *zeros: (lambda i: zeros)
    return pl.pallas_call(
        _block,
        grid=(n // _T,),
        in_specs=[
            pl.BlockSpec((_T, _D), lambda i: (i, 0)),
            pl.BlockSpec((_D, _NE), const(0, 0)),
            pl.BlockSpec((1, _NE), const(0, 0)),
            pl.BlockSpec((_NE, _D, _D), const(0, 0, 0)),
            pl.BlockSpec((_NE, _D), const(0, 0)),
            pl.BlockSpec((_D, _D), const(0, 0)),
            pl.BlockSpec((1, _D), const(0, 0)),
            pl.BlockSpec((_D, _D), const(0, 0)),
            pl.BlockSpec((1, _D), const(0, 0)),
            pl.BlockSpec((_D, _D), const(0, 0)),
            pl.BlockSpec((1, _D), const(0, 0)),
            pl.BlockSpec((_D, _D), const(0, 0)),
            pl.BlockSpec((1, _D), const(0, 0)),
        ],
        out_specs=pl.BlockSpec((_T, _D), lambda i: (i, 0)),
        out_shape=jax.ShapeDtypeStruct((n, _D), jnp.float32),
        compiler_params=pltpu.CompilerParams(
            dimension_semantics=("parallel",)),
    )(inputs, Wr, br2, We, be, Wq, bq2,
      Wk, bk2, Wv, bv2, Wo, bo2)
